# baseline (speedup 1.0000x reference)
import functools

import jax
import jax.numpy as jnp
from jax import lax
from jax.experimental import pallas as pl
from jax.experimental.pallas import tpu as pltpu
from jax.experimental.pallas import tpu_sc as plsc

_CLS = 1000
_CLS_PAD = 1024
_FEAT = 512
_BATCH = 16384
_NC, _NS = 2, 16
_NW = _NC * _NS
_BPW = _BATCH // _NW
_BPH = (_BATCH // 2) // _NW
_CHUNK = 64
_NCHUNK = _BPW // _CHUNK
_HYS = _BATCH // _NS


def _make_sc_body(first):
    nchunk = _BPH // _CHUNK

    def body(*refs):
        if first:
            (center_hbm, ys_hbm, ce_hbm, cnt_hbm, tbl_hbm,
             hidx2_v, idx_v, ones_v, cnt1024_v, cntout_v, rows_a, rows_b,
             sh_cnt, sem_a, sem_b) = refs
        else:
            (center_hbm, ys_hbm, tbl_hbm, ce_hbm, cnt_hbm,
             hidx2_v, idx_v, ones_v, cnt1024_v, cntout_v, rows_a, rows_b,
             sh_cnt, sem_a, sem_b) = refs
        c = lax.axis_index("c")
        s = lax.axis_index("s")
        wid = c * _NS + s
        base = (0 if first else _BATCH // 2) + wid * _BPH

        zero16 = jnp.zeros((16,), jnp.float32)
        ones16 = jnp.ones((16,), jnp.float32)

        pltpu.sync_copy(ys_hbm.at[pl.ds(base, _BPH)], idx_v)
        bufs = (rows_a, rows_b)
        sems = (sem_a, sem_b)
        cps = [None] * nchunk
        for ch in range(2):
            cps[ch] = pltpu.async_copy(
                center_hbm.at[idx_v.at[pl.ds(ch * _CHUNK, _CHUNK)]],
                bufs[ch], sems[ch])

        if first:
            for j in range(8):
                ones_v[pl.ds(j * 16, 16)] = ones16
            for j in range(_CLS_PAD // 16):
                cnt1024_v[pl.ds(j * 16, 16)] = zero16
            @pl.when(s == 0)
            def _():
                pltpu.sync_copy(cnt1024_v, sh_cnt)
            plsc.subcore_barrier()
            for j in range(_HYS // 128):
                pltpu.sync_copy(ys_hbm.at[pl.ds(s * _HYS + j * 128, 128)],
                                hidx2_v.at[j])
            for j in range(_HYS // 128):
                pltpu.sync_copy(ones_v, sh_cnt.at[hidx2_v.at[j]], add=True)
            plsc.subcore_barrier()
            pltpu.sync_copy(sh_cnt, cnt1024_v)
            @pl.when(wid == 0)
            def _():
                pltpu.sync_copy(cnt1024_v, tbl_hbm)
        else:
            pltpu.sync_copy(tbl_hbm, cnt1024_v)

        for k in range(_BPH // 16):
            i16 = idx_v[pl.ds(k * 16, 16)]
            cntout_v[pl.ds(k * 16, 16)] = plsc.load_gather(cnt1024_v, [i16])
        pltpu.sync_copy(cntout_v, cnt_hbm.at[pl.ds(base - (0 if first else _BATCH // 2), _BPH)])

        out_base = wid * _BPH
        for ch in range(nchunk):
            cps[ch].wait()
            pltpu.sync_copy(bufs[ch % 2],
                            ce_hbm.at[pl.ds(out_base + ch * _CHUNK, _CHUNK)])
            if ch + 2 < nchunk:
                cps[ch + 2] = pltpu.async_copy(
                    center_hbm.at[idx_v.at[pl.ds((ch + 2) * _CHUNK, _CHUNK)]],
                    bufs[ch % 2], sems[ch % 2])

    return body


def _half_scratch():
    return [
        pltpu.VMEM((_HYS // 128, 128), jnp.int32),
        pltpu.VMEM((_BPH,), jnp.int32),
        pltpu.VMEM((128,), jnp.float32),
        pltpu.VMEM((_CLS_PAD,), jnp.float32),
        pltpu.VMEM((_BPH,), jnp.float32),
        pltpu.VMEM((_CHUNK, _FEAT // 2), jnp.int32),
        pltpu.VMEM((_CHUNK, _FEAT // 2), jnp.int32),
        pltpu.VMEM_SHARED((_CLS_PAD,), jnp.float32),
        pltpu.SemaphoreType.DMA,
        pltpu.SemaphoreType.DMA,
    ]


@functools.lru_cache(maxsize=1)
def _make_sc_first():
    return pl.kernel(
        _make_sc_body(True),
        out_type=[jax.ShapeDtypeStruct((_BATCH // 2, _FEAT // 2), jnp.int32),
                  jax.ShapeDtypeStruct((_BATCH // 2,), jnp.float32),
                  jax.ShapeDtypeStruct((_CLS_PAD,), jnp.float32)],
        mesh=plsc.VectorSubcoreMesh(core_axis_name="c",
                                    subcore_axis_name="s"),
        scratch_types=_half_scratch(),
        compiler_params=pltpu.CompilerParams(needs_layout_passes=False),
    )


@functools.lru_cache(maxsize=1)
def _make_sc_second():
    return pl.kernel(
        _make_sc_body(False),
        out_type=[jax.ShapeDtypeStruct((_BATCH // 2, _FEAT // 2), jnp.int32),
                  jax.ShapeDtypeStruct((_BATCH // 2,), jnp.float32)],
        mesh=plsc.VectorSubcoreMesh(core_axis_name="c",
                                    subcore_axis_name="s"),
        scratch_types=_half_scratch(),
        compiler_params=pltpu.CompilerParams(needs_layout_passes=False),
    )


_BT = 4096


def _tc_body(xs_ref, cew_ref, cnt_ref, out_ref):
    i = pl.program_id(0)
    x = xs_ref[...]
    w = cew_ref[...]
    g_lo = lax.bitcast_convert_type(w << 16, jnp.float32)
    g_hi = lax.bitcast_convert_type(w & jnp.int32(-65536), jnp.float32)
    n2 = jnp.sum(x * x, axis=1, keepdims=True)
    nrm = jnp.maximum(jnp.sqrt(n2), 1e-12)
    dl = x[:, :_FEAT // 2] / nrm - g_lo
    dh = x[:, _FEAT // 2:] / nrm - g_hi
    d2 = (jnp.sum(dl * dl, axis=1, keepdims=True)
          + jnp.sum(dh * dh, axis=1, keepdims=True))
    dist = jnp.sqrt(d2)
    blk = jnp.sum(dist / cnt_ref[...])

    @pl.when(i == 0)
    def _():
        out_ref[0, 0] = 0.0

    out_ref[0, 0] += blk


def _tc_loss(xs, ce_w, cnt):
    n = xs.shape[0]
    out = pl.pallas_call(
        _tc_body,
        grid=(n // _BT,),
        in_specs=[
            pl.BlockSpec((_BT, _FEAT), lambda i: (i, 0)),
            pl.BlockSpec((_BT, _FEAT // 2), lambda i: (i, 0)),
            pl.BlockSpec((_BT, 1), lambda i: (i, 0)),
        ],
        out_specs=pl.BlockSpec((1, 1), lambda i: (0, 0),
                               memory_space=pltpu.SMEM),
        out_shape=jax.ShapeDtypeStruct((1, 1), jnp.float32),
    )(xs, ce_w, cnt.reshape(n, 1))
    return out[0, 0]


def kernel(xs, ys, center):
    ys32 = ys.astype(jnp.int32)
    half = _FEAT // 2
    lo16 = lax.bitcast_convert_type(
        center[:, :half].astype(jnp.bfloat16), jnp.uint16).astype(jnp.uint32)
    hi16 = lax.bitcast_convert_type(
        center[:, half:].astype(jnp.bfloat16), jnp.uint16).astype(jnp.uint32)
    table = lax.bitcast_convert_type((hi16 << 16) | lo16, jnp.int32)
    ce1, cnt1, tbl = _make_sc_first()(table, ys32)
    ce2, cnt2 = _make_sc_second()(table, ys32, tbl)
    hb = _BATCH // 2
    l1 = _tc_loss(xs[:hb], ce1, cnt1)
    l2 = _tc_loss(xs[hb:], ce2, cnt2)
    return l1 + l2


# speedup vs baseline: 1.4481x; 1.4481x over previous
import functools

import jax
import jax.numpy as jnp
from jax import lax
from jax.experimental import pallas as pl
from jax.experimental.pallas import tpu as pltpu
from jax.experimental.pallas import tpu_sc as plsc

_CLS = 1000
_CLS_PAD = 1024
_FEAT = 512
_BATCH = 16384
_NC, _NS = 2, 16
_NW = _NC * _NS
_BPW = _BATCH // _NW
_CHUNK = 128
_NCHUNK = _BPW // _CHUNK
_HYS = _BATCH // _NS


def _sc_body(center_hbm, ys_hbm, ce_hbm, cnt_hbm,
             hidx2_v, idx_v, ones_v, cnt1024_v, cntout_v, rows_a, rows_b,
             sh_cnt, sem_a, sem_b):
    c = lax.axis_index("c")
    s = lax.axis_index("s")
    wid = c * _NS + s
    base = wid * _BPW

    zero16 = jnp.zeros((16,), jnp.float32)
    ones16 = jnp.ones((16,), jnp.float32)
    for j in range(8):
        ones_v[pl.ds(j * 16, 16)] = ones16
    for j in range(_CLS_PAD // 16):
        cnt1024_v[pl.ds(j * 16, 16)] = zero16

    pltpu.sync_copy(ys_hbm.at[pl.ds(base, _BPW)], idx_v)
    bufs = (rows_a, rows_b)
    sems = (sem_a, sem_b)
    cps = [None] * _NCHUNK
    for ch in range(2):
        cps[ch] = pltpu.async_copy(
            center_hbm.at[idx_v.at[pl.ds(ch * _CHUNK, _CHUNK)]],
            bufs[ch], sems[ch])

    @pl.when(s == 0)
    def _():
        pltpu.sync_copy(cnt1024_v, sh_cnt)
    plsc.subcore_barrier()
    for j in range(_HYS // 128):
        pltpu.sync_copy(ys_hbm.at[pl.ds(s * _HYS + j * 128, 128)],
                        hidx2_v.at[j])
    for j in range(_HYS // 128):
        pltpu.sync_copy(ones_v, sh_cnt.at[hidx2_v.at[j]], add=True)
    plsc.subcore_barrier()

    pltpu.sync_copy(sh_cnt, cnt1024_v)
    for k in range(_BPW // 16):
        i16 = idx_v[pl.ds(k * 16, 16)]
        cntout_v[pl.ds(k * 16, 16)] = plsc.load_gather(cnt1024_v, [i16])
    pltpu.sync_copy(cntout_v, cnt_hbm.at[pl.ds(base, _BPW)])

    for ch in range(_NCHUNK):
        cps[ch].wait()
        pltpu.sync_copy(bufs[ch % 2],
                        ce_hbm.at[pl.ds(base + ch * _CHUNK, _CHUNK)])
        if ch + 2 < _NCHUNK:
            cps[ch + 2] = pltpu.async_copy(
                center_hbm.at[idx_v.at[pl.ds((ch + 2) * _CHUNK, _CHUNK)]],
                bufs[ch % 2], sems[ch % 2])


@functools.lru_cache(maxsize=1)
def _make_sc_gather():
    return pl.kernel(
        _sc_body,
        out_type=[jax.ShapeDtypeStruct((_BATCH, _FEAT // 2), jnp.int32),
                  jax.ShapeDtypeStruct((_BATCH,), jnp.float32)],
        mesh=plsc.VectorSubcoreMesh(core_axis_name="c",
                                    subcore_axis_name="s"),
        scratch_types=[
            pltpu.VMEM((_HYS // 128, 128), jnp.int32),
            pltpu.VMEM((_BPW,), jnp.int32),
            pltpu.VMEM((128,), jnp.float32),
            pltpu.VMEM((_CLS_PAD,), jnp.float32),
            pltpu.VMEM((_BPW,), jnp.float32),
            pltpu.VMEM((_CHUNK, _FEAT // 2), jnp.int32),
            pltpu.VMEM((_CHUNK, _FEAT // 2), jnp.int32),
            pltpu.VMEM_SHARED((_CLS_PAD,), jnp.float32),
            pltpu.SemaphoreType.DMA,
            pltpu.SemaphoreType.DMA,
        ],
        compiler_params=pltpu.CompilerParams(needs_layout_passes=False),
    )


_BT = 4096


def _tc_body(xs_ref, cew_ref, cnt_ref, out_ref):
    i = pl.program_id(0)
    x = xs_ref[...]
    w = cew_ref[...]
    g_lo = lax.bitcast_convert_type(w << 16, jnp.float32)
    g_hi = lax.bitcast_convert_type(w & jnp.int32(-65536), jnp.float32)
    n2 = jnp.sum(x * x, axis=1, keepdims=True)
    nrm = jnp.maximum(jnp.sqrt(n2), 1e-12)
    dl = x[:, :_FEAT // 2] / nrm - g_lo
    dh = x[:, _FEAT // 2:] / nrm - g_hi
    d2 = (jnp.sum(dl * dl, axis=1, keepdims=True)
          + jnp.sum(dh * dh, axis=1, keepdims=True))
    dist = jnp.sqrt(d2)
    blk = jnp.sum(dist / cnt_ref[...])

    @pl.when(i == 0)
    def _():
        out_ref[0, 0] = 0.0

    out_ref[0, 0] += blk


def _tc_loss(xs, ce_w, cnt):
    out = pl.pallas_call(
        _tc_body,
        grid=(_BATCH // _BT,),
        in_specs=[
            pl.BlockSpec((_BT, _FEAT), lambda i: (i, 0)),
            pl.BlockSpec((_BT, _FEAT // 2), lambda i: (i, 0)),
            pl.BlockSpec((_BT, 1), lambda i: (i, 0)),
        ],
        out_specs=pl.BlockSpec((1, 1), lambda i: (0, 0),
                               memory_space=pltpu.SMEM),
        out_shape=jax.ShapeDtypeStruct((1, 1), jnp.float32),
    )(xs, ce_w, cnt.reshape(_BATCH, 1))
    return out[0, 0]


def kernel(xs, ys, center):
    ys32 = ys.astype(jnp.int32)
    half = _FEAT // 2
    lo16 = lax.bitcast_convert_type(
        center[:, :half].astype(jnp.bfloat16), jnp.uint16).astype(jnp.uint32)
    hi16 = lax.bitcast_convert_type(
        center[:, half:].astype(jnp.bfloat16), jnp.uint16).astype(jnp.uint32)
    table = lax.bitcast_convert_type((hi16 << 16) | lo16, jnp.int32)
    ce_w, cnt = _make_sc_gather()(table, ys32)
    return _tc_loss(xs, ce_w, cnt)
